# baseline (speedup 1.0000x reference)
import functools

import jax
import jax.numpy as jnp
from jax import lax
from jax.experimental import pallas as pl
from jax.experimental.pallas import tpu as pltpu
from jax.experimental.pallas import tpu_sc as plsc

_NC = 2
_NS = 16
_NW = _NC * _NS

_TB = 1024
_CH = 16


def _means_body(nblk, B, TB, rs_ref, data_ref, out_ref):
    k = pl.program_id(0)
    row = lax.broadcasted_iota(jnp.int32, (1, TB), 1) + k * TB
    rows = []
    for b in range(B):
        lo = rs_ref[b]
        hi = rs_ref[b + 1]
        inv = 1.0 / (hi - lo).astype(jnp.float32)
        inb = (row >= lo) & (row < hi)
        rows.append(jnp.where(inb, inv, 0.0))
    oh = jnp.concatenate(rows, axis=0)
    part = jnp.dot(oh, data_ref[...], preferred_element_type=jnp.float32)

    @pl.when(k == 0)
    def _init():
        out_ref[...] = jnp.zeros_like(out_ref)

    out_ref[...] += part


def _segment_means(data, rs):
    T, D = data.shape
    B = rs.shape[0] - 1
    nblk = T // _TB
    return pl.pallas_call(
        functools.partial(_means_body, nblk, B, _TB),
        grid=(nblk,),
        in_specs=[
            pl.BlockSpec(memory_space=pltpu.SMEM),
            pl.BlockSpec((_TB, D), lambda k: (k, 0)),
        ],
        out_specs=pl.BlockSpec((B, D), lambda k: (0, 0)),
        out_shape=jax.ShapeDtypeStruct((B, D), jnp.float32),
    )(rs, data)


def _gather_means(means, idx3, T, D):
    B = means.shape[0]
    PW = T // _NW
    G = 16
    NG = PW // G
    mesh = plsc.VectorSubcoreMesh(core_axis_name="c", subcore_axis_name="s")

    @functools.partial(
        pl.kernel,
        mesh=mesh,
        out_type=jax.ShapeDtypeStruct((T, D), jnp.float32),
        scratch_types=[
            pltpu.VMEM((NG, G), jnp.int32),
            pltpu.VMEM((B, D), jnp.float32),
            pltpu.SemaphoreType.DMA,
        ],
    )
    def body(means_hbm, idx_hbm, out_hbm, idx_v, means_v, sem0):
        wid = lax.axis_index("s") * _NC + lax.axis_index("c")
        base = wid * PW
        pltpu.sync_copy(idx_hbm.at[wid], idx_v)
        pltpu.sync_copy(means_hbm, means_v)

        def issue_group(g):
            vg = idx_v[g]
            for j in range(G):
                s = vg[j]
                pltpu.async_copy(
                    means_v.at[s], out_hbm.at[base + g * G + j], sem0)

        def drain_group():
            for _ in range(G):
                pltpu.make_async_copy(
                    means_v.at[0], out_hbm.at[base], sem0).wait()

        def loop_body(g, carry):
            issue_group(g)

            @pl.when(g >= 2)
            def _():
                drain_group()

            return carry

        lax.fori_loop(0, NG, loop_body, 0)
        drain_group()
        drain_group()

    return body(means, idx3)


def kernel(data, rs, indices):
    T, D = data.shape
    idx = indices.reshape(-1).astype(jnp.int32)
    means = _segment_means(data, rs.astype(jnp.int32))
    PW = T // _NW
    idx3 = idx.reshape(_NW, PW // 16, 16)
    out = _gather_means(means, idx3, T, D)
    means2 = _segment_means(data, jnp.sort(rs.astype(jnp.int32)))
    return out, jnp.sum(means2)


# speedup vs baseline: 1.3744x; 1.3744x over previous
import functools

import jax
import jax.numpy as jnp
from jax import lax
from jax.experimental import pallas as pl
from jax.experimental.pallas import tpu as pltpu
from jax.experimental.pallas import tpu_sc as plsc

_NC = 2
_NS = 16
_NW = _NC * _NS

_TB = 1024
_CH = 16


def _means_body(nblk, B, TB, rs_ref, data_ref, out_ref):
    k = pl.program_id(0)
    row = lax.broadcasted_iota(jnp.int32, (1, TB), 1) + k * TB
    rows = []
    for b in range(B):
        lo = rs_ref[b]
        hi = rs_ref[b + 1]
        inv = 1.0 / (hi - lo).astype(jnp.float32)
        inb = (row >= lo) & (row < hi)
        rows.append(jnp.where(inb, inv, 0.0))
    oh = jnp.concatenate(rows, axis=0)
    part = jnp.dot(oh, data_ref[...], preferred_element_type=jnp.float32)

    @pl.when(k == 0)
    def _init():
        out_ref[...] = jnp.zeros_like(out_ref)

    out_ref[...] += part


def _segment_means(data, rs):
    T, D = data.shape
    B = rs.shape[0] - 1
    nblk = T // _TB
    return pl.pallas_call(
        functools.partial(_means_body, nblk, B, _TB),
        grid=(nblk,),
        in_specs=[
            pl.BlockSpec(memory_space=pltpu.SMEM),
            pl.BlockSpec((_TB, D), lambda k: (k, 0)),
        ],
        out_specs=pl.BlockSpec((B, D), lambda k: (0, 0)),
        out_shape=jax.ShapeDtypeStruct((B, D), jnp.float32),
    )(rs, data)


def _gather_means(means, idx3, T, D):
    B = means.shape[0]
    PW = T // _NW
    G = 16
    NG = PW // G
    mesh = plsc.VectorSubcoreMesh(core_axis_name="c", subcore_axis_name="s")

    @functools.partial(
        pl.kernel,
        mesh=mesh,
        out_type=jax.ShapeDtypeStruct((T, D), jnp.float32),
        scratch_types=[
            pltpu.VMEM((NG, G), jnp.int32),
            pltpu.VMEM((B, D), jnp.float32),
            pltpu.SemaphoreType.DMA,
        ],
    )
    def body(means_hbm, idx_hbm, out_hbm, idx_v, means_v, sem0):
        wid = lax.axis_index("s") * _NC + lax.axis_index("c")
        base = wid * PW
        pltpu.sync_copy(idx_hbm.at[wid], idx_v)
        pltpu.sync_copy(means_hbm, means_v)

        def issue_group(g):
            vg = idx_v[g]
            for j in range(G):
                s = vg[j]
                pltpu.async_copy(
                    means_v.at[s], out_hbm.at[base + g * G + j], sem0)

        def drain_group():
            for _ in range(G):
                pltpu.make_async_copy(
                    means_v.at[0], out_hbm.at[base], sem0).wait()

        def loop_body(g, carry):
            issue_group(g)

            @pl.when(g >= 3)
            def _():
                drain_group()

            return carry

        lax.fori_loop(0, NG, loop_body, 0)
        drain_group()
        drain_group()
        drain_group()

    return body(means, idx3)


def kernel(data, rs, indices):
    T, D = data.shape
    idx = indices.reshape(-1).astype(jnp.int32)
    means = _segment_means(data, rs.astype(jnp.int32))
    PW = T // _NW
    idx3 = idx.reshape(_NW, PW // 16, 16)
    return _gather_means(means, idx3, T, D)
